# baseline (speedup 1.0000x reference)
import functools

import jax
import jax.numpy as jnp
from jax import lax
from jax.experimental import pallas as pl
from jax.experimental.pallas import tpu as pltpu
from jax.experimental.pallas import tpu_sc as plsc

VOCAB = 100000
DIM = 64
BATCH = 16384

_INFO = plsc.get_sparse_core_info()
_NC, _NS = _INFO.num_cores, _INFO.num_subcores
_NW = _NC * _NS
_CHUNK = 128
_B_PER_W = BATCH // _NW
_CHUNKS_PER_W = _B_PER_W // _CHUNK


@functools.partial(
    pl.kernel,
    mesh=plsc.VectorSubcoreMesh(core_axis_name="c", subcore_axis_name="s"),
    out_type=jax.ShapeDtypeStruct((BATCH, DIM), jnp.float32),
    scratch_types=[
        pltpu.VMEM((_CHUNKS_PER_W, _CHUNK), jnp.int32),
        pltpu.VMEM((_B_PER_W, DIM), jnp.float32),
        pltpu.SemaphoreType.DMA,
    ],
    compiler_params=pltpu.CompilerParams(use_tc_tiling_on_sc=False),
)
def _gather_kernel(idx_hbm, table_hbm, out_hbm, idx_v, rows_v, sem):
    wid = lax.axis_index("s") * _NC + lax.axis_index("c")
    base = wid * _B_PER_W
    pltpu.sync_copy(idx_hbm.at[pl.ds(wid * _CHUNKS_PER_W, _CHUNKS_PER_W)], idx_v)
    copies = []
    for j in range(_CHUNKS_PER_W):
        copies.append(pltpu.async_copy(
            table_hbm.at[idx_v.at[jnp.int32(j)]],
            rows_v.at[pl.ds(j * _CHUNK, _CHUNK)],
            sem,
        ))
    for c in copies:
        c.wait()
    pltpu.sync_copy(rows_v, out_hbm.at[pl.ds(base, _B_PER_W)])


def kernel(input_tokens, table):
    idx = input_tokens.astype(jnp.int32).reshape(BATCH // _CHUNK, _CHUNK)
    return _gather_kernel(idx, table)


# speedup vs baseline: 1.8599x; 1.8599x over previous
import functools

import jax
import jax.numpy as jnp
import numpy as np
from jax import lax
from jax.experimental import pallas as pl
from jax.experimental.pallas import tpu as pltpu
from jax.experimental.pallas import tpu_sc as plsc

VOCAB = 100000
DIM = 64
BATCH = 16384

_INFO = plsc.get_sparse_core_info()
_NC, _NS = _INFO.num_cores, _INFO.num_subcores
_NW = _NC * _NS
_ROWS_PER_W = DIM // _NW
_TCHUNK = 2048
_NCHUNKS = BATCH // _TCHUNK
_L = 16


@functools.partial(
    pl.kernel,
    mesh=plsc.VectorSubcoreMesh(core_axis_name="c", subcore_axis_name="s"),
    out_type=jax.ShapeDtypeStruct((DIM, BATCH), jnp.float32),
    scratch_types=[
        pltpu.VMEM((BATCH,), jnp.int32),
        pltpu.VMEM((VOCAB,), jnp.float32),
        pltpu.VMEM((_TCHUNK,), jnp.float32),
        pltpu.SemaphoreType.DMA,
    ],
    compiler_params=pltpu.CompilerParams(needs_layout_passes=False),
)
def _gather_kernel(idx_hbm, tableT_hbm, outT_hbm, tok_v, row_v, obuf_v, sem):
    wid = lax.axis_index("s") * _NC + lax.axis_index("c")
    pltpu.sync_copy(idx_hbm, tok_v)
    for r in range(_ROWS_PER_W):
        d = wid * _ROWS_PER_W + r
        pltpu.sync_copy(tableT_hbm.at[d], row_v)
        for c in range(_NCHUNKS):
            def body(_, ooff):
                tok = tok_v[pl.ds(ooff + np.int32(c * _TCHUNK), _L)]
                obuf_v[pl.ds(ooff, _L)] = plsc.load_gather(row_v, [tok])
                return ooff + np.int32(_L)
            lax.fori_loop(np.int32(0), np.int32(_TCHUNK // _L), body,
                          np.int32(0))
            pltpu.sync_copy(obuf_v, outT_hbm.at[d, pl.ds(c * _TCHUNK, _TCHUNK)])


def kernel(input_tokens, table):
    idx = input_tokens.astype(jnp.int32)
    outT = _gather_kernel(idx, table.T)
    return outT.T


# speedup vs baseline: 2.7954x; 1.5030x over previous
import functools

import jax
import jax.numpy as jnp
import numpy as np
from jax import lax
from jax.experimental import pallas as pl
from jax.experimental.pallas import tpu as pltpu
from jax.experimental.pallas import tpu_sc as plsc

VOCAB = 100000
DIM = 64
BATCH = 16384

_INFO = plsc.get_sparse_core_info()
_NC, _NS = _INFO.num_cores, _INFO.num_subcores
_NW = _NC * _NS
_ROWS_PER_W = DIM // _NW
_TCHUNK = 4096
_NCHUNKS = BATCH // _TCHUNK
_L = 16


@functools.partial(
    pl.kernel,
    mesh=plsc.VectorSubcoreMesh(core_axis_name="c", subcore_axis_name="s"),
    out_type=jax.ShapeDtypeStruct((DIM, BATCH), jnp.float32),
    scratch_types=[
        pltpu.VMEM((BATCH,), jnp.int32),
        pltpu.VMEM((VOCAB,), jnp.float32),
        pltpu.VMEM((_TCHUNK,), jnp.float32),
        pltpu.VMEM((_TCHUNK,), jnp.float32),
        pltpu.SemaphoreType.DMA,
        pltpu.SemaphoreType.DMA,
        pltpu.SemaphoreType.DMA,
    ],
    compiler_params=pltpu.CompilerParams(needs_layout_passes=False),
)
def _gather_kernel(idx_hbm, tableT_hbm, outT_hbm, tok_v, row_v, ob0, ob1,
                   sem_t, sem_r, sem_o):
    wid = lax.axis_index("s") * _NC + lax.axis_index("c")
    d0 = wid * np.int32(_ROWS_PER_W)
    tok_cp = pltpu.async_copy(idx_hbm, tok_v, sem_t)
    row_cp = pltpu.async_copy(tableT_hbm.at[d0], row_v, sem_r)
    tok_cp.wait()
    obufs = (ob0, ob1)
    out_cps = []
    for r in range(_ROWS_PER_W):
        d = d0 + np.int32(r)
        row_cp.wait()
        for c in range(_NCHUNKS):
            ob = obufs[c % 2]
            if len(out_cps) >= 2:
                out_cps.pop(0).wait()
            base = np.int32(c * _TCHUNK)

            @plsc.parallel_loop(base, np.int32((c + 1) * _TCHUNK),
                                step=np.int32(_L), unroll=4)
            def body(i):
                tok = tok_v[pl.ds(i, _L)]
                ob[pl.ds(i - base, _L)] = plsc.load_gather(row_v, [tok])

            if r == 0 and c == _NCHUNKS - 1:
                row_cp = pltpu.async_copy(tableT_hbm.at[d0 + np.int32(1)],
                                          row_v, sem_r)
            out_cps.append(pltpu.async_copy(
                ob, outT_hbm.at[d, pl.ds(int(base), _TCHUNK)], sem_o))
    for cp in out_cps:
        cp.wait()


def kernel(input_tokens, table):
    idx = input_tokens.astype(jnp.int32)
    outT = _gather_kernel(idx, table.T)
    return outT.T


# speedup vs baseline: 3.0111x; 1.0772x over previous
import functools

import jax
import jax.numpy as jnp
import numpy as np
from jax import lax
from jax.experimental import pallas as pl
from jax.experimental.pallas import tpu as pltpu
from jax.experimental.pallas import tpu_sc as plsc

VOCAB = 100000
DIM = 64
BATCH = 16384

_INFO = plsc.get_sparse_core_info()
_NC, _NS = _INFO.num_cores, _INFO.num_subcores
_NW = _NC * _NS
_ROWS_PER_W = DIM // _NW
_TCHUNK = 4096
_NCHUNKS = BATCH // _TCHUNK
_L = 16


@functools.partial(
    pl.kernel,
    mesh=plsc.VectorSubcoreMesh(core_axis_name="c", subcore_axis_name="s"),
    out_type=jax.ShapeDtypeStruct((DIM, BATCH), jnp.float32),
    scratch_types=[
        pltpu.VMEM((BATCH,), jnp.int32),
        pltpu.VMEM((VOCAB,), jnp.float32),
        pltpu.VMEM((_TCHUNK,), jnp.float32),
        pltpu.VMEM((_TCHUNK,), jnp.float32),
        pltpu.VMEM_SHARED((BATCH,), jnp.int32),
        pltpu.SemaphoreType.DMA,
        pltpu.SemaphoreType.DMA,
        pltpu.SemaphoreType.DMA,
    ],
    compiler_params=pltpu.CompilerParams(needs_layout_passes=False),
)
def _gather_kernel(idx_hbm, tableT_hbm, outT_hbm, tok_v, row_v, ob0, ob1,
                   tok_sh, sem_t, sem_r, sem_o):
    sid = lax.axis_index("s")
    wid = sid * _NC + lax.axis_index("c")
    d0 = wid * np.int32(_ROWS_PER_W)
    row_cp = pltpu.async_copy(tableT_hbm.at[d0], row_v, sem_r)
    @pl.when(sid == np.int32(0))
    def _():
        pltpu.sync_copy(idx_hbm, tok_sh)
    plsc.subcore_barrier()
    pltpu.async_copy(tok_sh, tok_v, sem_t).wait()
    obufs = (ob0, ob1)
    out_cps = []
    for r in range(_ROWS_PER_W):
        d = d0 + np.int32(r)
        row_cp.wait()
        for c in range(_NCHUNKS):
            ob = obufs[c % 2]
            if len(out_cps) >= 2:
                out_cps.pop(0).wait()
            base = np.int32(c * _TCHUNK)

            @plsc.parallel_loop(base, np.int32((c + 1) * _TCHUNK),
                                step=np.int32(_L), unroll=8)
            def body(i):
                tok = tok_v[pl.ds(i, _L)]
                ob[pl.ds(i - base, _L)] = plsc.load_gather(row_v, [tok])

            if r == 0 and c == _NCHUNKS - 1:
                row_cp = pltpu.async_copy(tableT_hbm.at[d0 + np.int32(1)],
                                          row_v, sem_r)
            out_cps.append(pltpu.async_copy(
                ob, outT_hbm.at[d, pl.ds(int(base), _TCHUNK)], sem_o))
    for cp in out_cps:
        cp.wait()


def kernel(input_tokens, table):
    idx = input_tokens.astype(jnp.int32)
    outT = _gather_kernel(idx, table.T)
    return outT.T
